# baseline (speedup 1.0000x reference)
import jax
import jax.numpy as jnp
from jax.experimental import pallas as pl
from jax.experimental.pallas import tpu as pltpu

_NTOK = 16384
_HIDDEN = 2048
_NE = 64
_NPAD = 128
_BLK_M = 512


def _body(b_ref, x_ref, w_ref, logits_ref, probs_ref, cap_ref):
    x = x_ref[...]
    w = w_ref[...]
    acc = jnp.dot(x, w, preferred_element_type=jnp.float32)
    logits = acc[:, :_NE]
    logits_ref[...] = logits
    m = jnp.max(logits, axis=-1, keepdims=True)
    e = jnp.exp(logits - m)
    probs_ref[...] = e / jnp.sum(e, axis=-1, keepdims=True)
    cap_ref[...] = jax.nn.sigmoid(acc[:, _NE:_NE + 1] + b_ref[0])


def kernel(x, W_router, W_cap, b_cap):
    w_all = jnp.concatenate([W_router, W_cap], axis=0)
    w_all = jnp.pad(w_all, ((0, _NPAD - _NE - 1), (0, 0))).T

    grid = (_NTOK // _BLK_M,)
    out_shapes = (
        jax.ShapeDtypeStruct((_NTOK, _NE), jnp.float32),
        jax.ShapeDtypeStruct((_NTOK, _NE), jnp.float32),
        jax.ShapeDtypeStruct((_NTOK, 1), jnp.float32),
    )
    return pl.pallas_call(
        _body,
        grid_spec=pltpu.PrefetchScalarGridSpec(
            num_scalar_prefetch=1,
            grid=grid,
            in_specs=[
                pl.BlockSpec((_BLK_M, _HIDDEN), lambda i, b: (i, 0)),
                pl.BlockSpec((_HIDDEN, _NPAD), lambda i, b: (0, 0)),
            ],
            out_specs=[
                pl.BlockSpec((_BLK_M, _NE), lambda i, b: (i, 0)),
                pl.BlockSpec((_BLK_M, _NE), lambda i, b: (i, 0)),
                pl.BlockSpec((_BLK_M, 1), lambda i, b: (i, 0)),
            ],
        ),
        out_shape=out_shapes,
        compiler_params=pltpu.CompilerParams(
            dimension_semantics=("arbitrary",),
        ),
    )(b_cap, x, w_all)


# speedup vs baseline: 1.1014x; 1.1014x over previous
import jax
import jax.numpy as jnp
from jax.experimental import pallas as pl
from jax.experimental.pallas import tpu as pltpu

_NTOK = 16384
_HIDDEN = 2048
_NE = 64
_NPAD = 128
_BLK_M = 1024


def _body(b_ref, x_ref, w_ref, logits_ref, probs_ref, cap_ref):
    x = x_ref[...]
    w = w_ref[...]
    acc = jnp.dot(x, w, preferred_element_type=jnp.float32)
    logits = acc[:, :_NE]
    logits_ref[...] = logits
    m = jnp.max(logits, axis=-1, keepdims=True)
    e = jnp.exp(logits - m)
    probs_ref[...] = e / jnp.sum(e, axis=-1, keepdims=True)
    cap_ref[...] = jax.nn.sigmoid(acc[:, _NE:_NE + 1] + b_ref[0])


def kernel(x, W_router, W_cap, b_cap):
    w_all = jnp.concatenate([W_router, W_cap], axis=0)
    w_all = jnp.pad(w_all, ((0, _NPAD - _NE - 1), (0, 0))).T

    grid = (_NTOK // _BLK_M,)
    out_shapes = (
        jax.ShapeDtypeStruct((_NTOK, _NE), jnp.float32),
        jax.ShapeDtypeStruct((_NTOK, _NE), jnp.float32),
        jax.ShapeDtypeStruct((_NTOK, 1), jnp.float32),
    )
    return pl.pallas_call(
        _body,
        grid_spec=pltpu.PrefetchScalarGridSpec(
            num_scalar_prefetch=1,
            grid=grid,
            in_specs=[
                pl.BlockSpec((_BLK_M, _HIDDEN), lambda i, b: (i, 0)),
                pl.BlockSpec((_HIDDEN, _NPAD), lambda i, b: (0, 0)),
            ],
            out_specs=[
                pl.BlockSpec((_BLK_M, _NE), lambda i, b: (i, 0)),
                pl.BlockSpec((_BLK_M, _NE), lambda i, b: (i, 0)),
                pl.BlockSpec((_BLK_M, 1), lambda i, b: (i, 0)),
            ],
        ),
        out_shape=out_shapes,
        compiler_params=pltpu.CompilerParams(
            dimension_semantics=("arbitrary",),
        ),
    )(b_cap, x, w_all)


# speedup vs baseline: 1.1881x; 1.0788x over previous
import jax
import jax.numpy as jnp
from jax.experimental import pallas as pl
from jax.experimental.pallas import tpu as pltpu

_NTOK = 16384
_HIDDEN = 2048
_NE = 64
_NPAD = 128
_BLK_M = 2048


def _body(b_ref, x_ref, w_ref, logits_ref, probs_ref, cap_ref):
    x = x_ref[...]
    w = w_ref[...]
    acc = jnp.dot(x, w, preferred_element_type=jnp.float32)
    logits = acc[:, :_NE]
    logits_ref[...] = logits
    m = jnp.max(logits, axis=-1, keepdims=True)
    e = jnp.exp(logits - m)
    probs_ref[...] = e / jnp.sum(e, axis=-1, keepdims=True)
    cap_ref[...] = jax.nn.sigmoid(acc[:, _NE:_NE + 1] + b_ref[0])


def kernel(x, W_router, W_cap, b_cap):
    w_all = jnp.concatenate([W_router, W_cap], axis=0)
    w_all = jnp.pad(w_all, ((0, _NPAD - _NE - 1), (0, 0))).T

    grid = (_NTOK // _BLK_M,)
    out_shapes = (
        jax.ShapeDtypeStruct((_NTOK, _NE), jnp.float32),
        jax.ShapeDtypeStruct((_NTOK, _NE), jnp.float32),
        jax.ShapeDtypeStruct((_NTOK, 1), jnp.float32),
    )
    return pl.pallas_call(
        _body,
        grid_spec=pltpu.PrefetchScalarGridSpec(
            num_scalar_prefetch=1,
            grid=grid,
            in_specs=[
                pl.BlockSpec((_BLK_M, _HIDDEN), lambda i, b: (i, 0)),
                pl.BlockSpec((_HIDDEN, _NPAD), lambda i, b: (0, 0)),
            ],
            out_specs=[
                pl.BlockSpec((_BLK_M, _NE), lambda i, b: (i, 0)),
                pl.BlockSpec((_BLK_M, _NE), lambda i, b: (i, 0)),
                pl.BlockSpec((_BLK_M, 1), lambda i, b: (i, 0)),
            ],
        ),
        out_shape=out_shapes,
        compiler_params=pltpu.CompilerParams(
            dimension_semantics=("arbitrary",),
        ),
    )(b_cap, x, w_all)


# speedup vs baseline: 1.2067x; 1.0157x over previous
import jax
import jax.numpy as jnp
from jax.experimental import pallas as pl
from jax.experimental.pallas import tpu as pltpu

_NTOK = 16384
_HIDDEN = 2048
_NE = 64
_NPAD = 128
_BLK_M = 2048
_HALF = _BLK_M // 2


def _head(acc, b, logits_ref, probs_ref, cap_ref, rows):
    logits = acc[:, :_NE]
    logits_ref[rows, :] = logits
    m = jnp.max(logits, axis=-1, keepdims=True)
    e = jnp.exp(logits - m)
    probs_ref[rows, :] = e / jnp.sum(e, axis=-1, keepdims=True)
    cap_ref[rows, :] = jax.nn.sigmoid(acc[:, _NE:_NE + 1] + b)


def _body(b_ref, xa_ref, xb_ref, w_ref, logits_ref, probs_ref, cap_ref):
    w = w_ref[...]
    b = b_ref[0]
    acc_a = jnp.dot(xa_ref[...], w, preferred_element_type=jnp.float32)
    _head(acc_a, b, logits_ref, probs_ref, cap_ref, pl.ds(0, _HALF))
    acc_b = jnp.dot(xb_ref[...], w, preferred_element_type=jnp.float32)
    _head(acc_b, b, logits_ref, probs_ref, cap_ref, pl.ds(_HALF, _HALF))


def kernel(x, W_router, W_cap, b_cap):
    w_all = jnp.concatenate([W_router, W_cap], axis=0)
    w_all = jnp.pad(w_all, ((0, _NPAD - _NE - 1), (0, 0))).T

    grid = (_NTOK // _BLK_M,)
    out_shapes = (
        jax.ShapeDtypeStruct((_NTOK, _NE), jnp.float32),
        jax.ShapeDtypeStruct((_NTOK, _NE), jnp.float32),
        jax.ShapeDtypeStruct((_NTOK, 1), jnp.float32),
    )
    return pl.pallas_call(
        _body,
        grid_spec=pltpu.PrefetchScalarGridSpec(
            num_scalar_prefetch=1,
            grid=grid,
            in_specs=[
                pl.BlockSpec((_HALF, _HIDDEN), lambda i, b: (2 * i, 0)),
                pl.BlockSpec((_HALF, _HIDDEN), lambda i, b: (2 * i + 1, 0)),
                pl.BlockSpec((_HIDDEN, _NPAD), lambda i, b: (0, 0)),
            ],
            out_specs=[
                pl.BlockSpec((_BLK_M, _NE), lambda i, b: (i, 0)),
                pl.BlockSpec((_BLK_M, _NE), lambda i, b: (i, 0)),
                pl.BlockSpec((_BLK_M, 1), lambda i, b: (i, 0)),
            ],
        ),
        out_shape=out_shapes,
        compiler_params=pltpu.CompilerParams(
            dimension_semantics=("arbitrary",),
        ),
    )(b_cap, x, x, w_all)


# speedup vs baseline: 1.2430x; 1.0300x over previous
import jax
import jax.numpy as jnp
from jax.experimental import pallas as pl
from jax.experimental.pallas import tpu as pltpu

_NTOK = 16384
_HIDDEN = 2048
_NE = 64
_NPAD = 128
_BLK_M = 2048
_NSPLIT = 4
_PART = _BLK_M // _NSPLIT


def _head(acc, b, logits_ref, probs_ref, cap_ref, rows):
    logits = acc[:, :_NE]
    logits_ref[rows, :] = logits
    m = jnp.max(logits, axis=-1, keepdims=True)
    e = jnp.exp(logits - m)
    probs_ref[rows, :] = e / jnp.sum(e, axis=-1, keepdims=True)
    cap_ref[rows, :] = jax.nn.sigmoid(acc[:, _NE:_NE + 1] + b)


def _body(b_ref, *refs):
    x_refs = refs[:_NSPLIT]
    w_ref, logits_ref, probs_ref, cap_ref = refs[_NSPLIT:]
    w = w_ref[...]
    b = b_ref[0]
    for p in range(_NSPLIT):
        acc = jnp.dot(x_refs[p][...], w, preferred_element_type=jnp.float32)
        _head(acc, b, logits_ref, probs_ref, cap_ref, pl.ds(p * _PART, _PART))


def kernel(x, W_router, W_cap, b_cap):
    w_all = jnp.concatenate([W_router, W_cap], axis=0)
    w_all = jnp.pad(w_all, ((0, _NPAD - _NE - 1), (0, 0))).T

    grid = (_NTOK // _BLK_M,)
    out_shapes = (
        jax.ShapeDtypeStruct((_NTOK, _NE), jnp.float32),
        jax.ShapeDtypeStruct((_NTOK, _NE), jnp.float32),
        jax.ShapeDtypeStruct((_NTOK, 1), jnp.float32),
    )

    def part_spec(p):
        return pl.BlockSpec(
            (_PART, _HIDDEN),
            lambda i, b, p=p: (_NSPLIT * i + p, 0),
        )

    return pl.pallas_call(
        _body,
        grid_spec=pltpu.PrefetchScalarGridSpec(
            num_scalar_prefetch=1,
            grid=grid,
            in_specs=[part_spec(p) for p in range(_NSPLIT)] + [
                pl.BlockSpec((_HIDDEN, _NPAD), lambda i, b: (0, 0)),
            ],
            out_specs=[
                pl.BlockSpec((_BLK_M, _NE), lambda i, b: (i, 0)),
                pl.BlockSpec((_BLK_M, _NE), lambda i, b: (i, 0)),
                pl.BlockSpec((_BLK_M, 1), lambda i, b: (i, 0)),
            ],
        ),
        out_shape=out_shapes,
        compiler_params=pltpu.CompilerParams(
            dimension_semantics=("arbitrary",),
        ),
    )(b_cap, *([x] * _NSPLIT), w_all)


# speedup vs baseline: 1.2533x; 1.0083x over previous
import jax
import jax.numpy as jnp
from jax.experimental import pallas as pl
from jax.experimental.pallas import tpu as pltpu

_NTOK = 16384
_HIDDEN = 2048
_NE = 64
_NPAD = 128
_BLK_M = 2048
_NSPLIT = 8
_PART = _BLK_M // _NSPLIT


def _head(acc, b, logits_ref, probs_ref, cap_ref, rows):
    logits = acc[:, :_NE]
    logits_ref[rows, :] = logits
    m = jnp.max(logits, axis=-1, keepdims=True)
    e = jnp.exp(logits - m)
    probs_ref[rows, :] = e / jnp.sum(e, axis=-1, keepdims=True)
    cap_ref[rows, :] = jax.nn.sigmoid(acc[:, _NE:_NE + 1] + b)


def _body(b_ref, *refs):
    x_refs = refs[:_NSPLIT]
    w_ref, logits_ref, probs_ref, cap_ref = refs[_NSPLIT:]
    w = w_ref[...]
    b = b_ref[0]
    for p in range(_NSPLIT):
        acc = jnp.dot(x_refs[p][...], w, preferred_element_type=jnp.float32)
        _head(acc, b, logits_ref, probs_ref, cap_ref, pl.ds(p * _PART, _PART))


def kernel(x, W_router, W_cap, b_cap):
    w_all = jnp.concatenate([W_router, W_cap], axis=0)
    w_all = jnp.pad(w_all, ((0, _NPAD - _NE - 1), (0, 0))).T

    grid = (_NTOK // _BLK_M,)
    out_shapes = (
        jax.ShapeDtypeStruct((_NTOK, _NE), jnp.float32),
        jax.ShapeDtypeStruct((_NTOK, _NE), jnp.float32),
        jax.ShapeDtypeStruct((_NTOK, 1), jnp.float32),
    )

    def part_spec(p):
        return pl.BlockSpec(
            (_PART, _HIDDEN),
            lambda i, b, p=p: (_NSPLIT * i + p, 0),
        )

    return pl.pallas_call(
        _body,
        grid_spec=pltpu.PrefetchScalarGridSpec(
            num_scalar_prefetch=1,
            grid=grid,
            in_specs=[part_spec(p) for p in range(_NSPLIT)] + [
                pl.BlockSpec((_HIDDEN, _NPAD), lambda i, b: (0, 0)),
            ],
            out_specs=[
                pl.BlockSpec((_BLK_M, _NE), lambda i, b: (i, 0)),
                pl.BlockSpec((_BLK_M, _NE), lambda i, b: (i, 0)),
                pl.BlockSpec((_BLK_M, 1), lambda i, b: (i, 0)),
            ],
        ),
        out_shape=out_shapes,
        compiler_params=pltpu.CompilerParams(
            dimension_semantics=("arbitrary",),
        ),
    )(b_cap, *([x] * _NSPLIT), w_all)
